# Initial kernel scaffold; baseline (speedup 1.0000x reference)
import jax
import jax.numpy as jnp
from jax.experimental import pallas as pl


def kernel(X, E, emb_nodes, emb_edges, edge_index, W_e1, b_e1, W_e2, b_e2, W_h1, b_h1, W_h2, b_h2):
    raise NotImplementedError("write your pallas kernel here")


# speedup vs baseline: 4.0595x; 4.0595x over previous
import dataclasses

import jax
import jax.numpy as jnp
from jax import lax
from jax.experimental import pallas as pl
from jax.experimental.pallas import tpu as pltpu
from jax.experimental.pallas import tpu_sc as plsc

N_NODES = 10000
N_EDGES = 320000
D = 128

NC = 2
NS = 16
NW = NC * NS
L = 16
CHUNK = 128
NCHUNK = N_EDGES // CHUNK
SPAN = 80
NCHUNK_PAD = NW * SPAN
BE = 2000

_HIGHEST = jax.lax.Precision.HIGHEST


def _silu(x):
    return x * jax.nn.sigmoid(x)


def _tables_body(x_ref, w1a_ref, w1b_ref, ts_ref, td_ref):
    x = x_ref[...]
    ts_ref[...] = jnp.dot(x, w1a_ref[...], preferred_element_type=jnp.float32,
                          precision=_HIGHEST)
    td_ref[...] = jnp.dot(x, w1b_ref[...], preferred_element_type=jnp.float32,
                          precision=_HIGHEST)


def _make_tables(X, w1a, w1b):
    bn = 1000
    return pl.pallas_call(
        _tables_body,
        grid=(N_NODES // bn,),
        in_specs=[
            pl.BlockSpec((bn, D), lambda i: (i, 0)),
            pl.BlockSpec((D, D), lambda i: (0, 0)),
            pl.BlockSpec((D, D), lambda i: (0, 0)),
        ],
        out_specs=[
            pl.BlockSpec((bn, D), lambda i: (i, 0)),
            pl.BlockSpec((bn, D), lambda i: (i, 0)),
        ],
        out_shape=[
            jax.ShapeDtypeStruct((N_NODES, D), jnp.float32),
            jax.ShapeDtypeStruct((N_NODES, D), jnp.float32),
        ],
    )(X, w1a, w1b)


def _gather_body(ts_hbm, td_hbm, src_hbm, dst_hbm, ep_hbm,
                 gs_hbm, gd_hbm, n2_hbm,
                 idx_s, idx_d, rows_s, rows_d, emb_v, n2_v, sem_s, sem_d):
    wid = lax.axis_index("s") * NC + lax.axis_index("c")
    pltpu.sync_copy(src_hbm.at[pl.ds(wid * SPAN, SPAN)], idx_s)
    pltpu.sync_copy(dst_hbm.at[pl.ds(wid * SPAN, SPAN)], idx_d)
    pltpu.sync_copy(ep_hbm, emb_v)

    @pl.loop(0, SPAN)
    def _(j):
        c = wid * SPAN + j

        @pl.when(c < NCHUNK)
        def _():
            base = c * CHUNK
            cp_s = pltpu.async_copy(ts_hbm.at[idx_s.at[j]], rows_s, sem_s)
            cp_d = pltpu.async_copy(td_hbm.at[idx_d.at[j]], rows_d, sem_d)
            for k in range(CHUNK // L):
                i4s = idx_s[j, pl.ds(k * L, L)] * 4
                i4d = idx_d[j, pl.ds(k * L, L)] * 4
                n2 = None
                for comp in range(3):
                    es = plsc.load_gather(emb_v, [i4s + comp])
                    ed = plsc.load_gather(emb_v, [i4d + comp])
                    dd = ed - es
                    n2 = dd * dd if n2 is None else n2 + dd * dd
                plsc.store_scatter(
                    n2_v,
                    [jnp.arange(L, dtype=jnp.int32) + k * L,
                     jnp.full((L,), 0, jnp.int32)],
                    n2)
            pltpu.sync_copy(n2_v, n2_hbm.at[pl.ds(base, CHUNK)])
            cp_s.wait()
            pltpu.sync_copy(rows_s, gs_hbm.at[pl.ds(base, CHUNK)])
            cp_d.wait()
            pltpu.sync_copy(rows_d, gd_hbm.at[pl.ds(base, CHUNK)])


def _sc_compiler_params():
    cp = pltpu.CompilerParams()
    if "needs_layout_passes" in pltpu.CompilerParams.__dataclass_fields__:
        cp = dataclasses.replace(cp, needs_layout_passes=False)
    return cp


def _sc_gather(ts, td, src2d, dst2d, ep4):
    mesh = plsc.VectorSubcoreMesh(core_axis_name="c", subcore_axis_name="s",
                                  num_cores=NC, num_subcores=NS)
    kern = pl.kernel(
        _gather_body,
        compiler_params=_sc_compiler_params(),
        out_type=[
            jax.ShapeDtypeStruct((N_EDGES, D), jnp.float32),
            jax.ShapeDtypeStruct((N_EDGES, D), jnp.float32),
            jax.ShapeDtypeStruct((N_EDGES, 16), jnp.float32),
        ],
        mesh=mesh,
        scratch_types=[
            pltpu.VMEM((SPAN, CHUNK), jnp.int32),
            pltpu.VMEM((SPAN, CHUNK), jnp.int32),
            pltpu.VMEM((CHUNK, D), jnp.float32),
            pltpu.VMEM((CHUNK, D), jnp.float32),
            pltpu.VMEM((4 * N_NODES,), jnp.float32),
            pltpu.VMEM((CHUNK, 16), jnp.float32),
            pltpu.SemaphoreType.DMA,
            pltpu.SemaphoreType.DMA,
        ],
    )
    return kern(ts, td, src2d, dst2d, ep4)


def _edge_body(gs_ref, gd_ref, n2_ref, ee_ref, wn_ref, wd_ref, b1_ref,
               we2_ref, b2_ref, out_ref):
    ee = ee_ref[0]
    nrm = jnp.sqrt(n2_ref[:, 0:1])
    pre1 = (gs_ref[...] + gd_ref[...] + b1_ref[...]
            + nrm * wn_ref[...]
            + ee[:, 0:1] * wd_ref[0:1, :] + ee[:, 1:2] * wd_ref[1:2, :])
    t = _silu(pre1)
    pre2 = jnp.dot(t, we2_ref[...], preferred_element_type=jnp.float32,
                   precision=_HIGHEST) + b2_ref[...]
    out_ref[...] = _silu(pre2)


def _edge_mlp(gs, gd, n2, ee3, wn, wd, b1, we2, b2):
    return pl.pallas_call(
        _edge_body,
        grid=(N_EDGES // BE,),
        in_specs=[
            pl.BlockSpec((BE, D), lambda i: (i, 0)),
            pl.BlockSpec((BE, D), lambda i: (i, 0)),
            pl.BlockSpec((BE, 16), lambda i: (i, 0)),
            pl.BlockSpec((1, BE, 2), lambda i: (i, 0, 0)),
            pl.BlockSpec((1, D), lambda i: (0, 0)),
            pl.BlockSpec((2, D), lambda i: (0, 0)),
            pl.BlockSpec((1, D), lambda i: (0, 0)),
            pl.BlockSpec((D, D), lambda i: (0, 0)),
            pl.BlockSpec((1, D), lambda i: (0, 0)),
        ],
        out_specs=pl.BlockSpec((BE, D), lambda i: (i, 0)),
        out_shape=jax.ShapeDtypeStruct((N_EDGES, D), jnp.float32),
    )(gs, gd, n2, ee3, wn, wd, b1, we2, b2)


ACC_ROWS = N_NODES


def _scatter_body(mij_hbm, dst_hbm, zeros_hbm, p_hbm, idx_v, rows_v, acc, sem):
    cid = lax.axis_index("c")
    sid = lax.axis_index("s")

    @pl.when(sid == 0)
    def _():
        pltpu.sync_copy(zeros_hbm, acc)

    wid = sid * NC + cid
    pltpu.sync_copy(dst_hbm.at[pl.ds(wid * SPAN, SPAN)], idx_v)
    plsc.subcore_barrier()

    @pl.loop(0, SPAN)
    def _(j):
        c = wid * SPAN + j

        @pl.when(c < NCHUNK)
        def _():
            pltpu.async_copy(mij_hbm.at[pl.ds(c * CHUNK, CHUNK)], rows_v,
                             sem).wait()
            pltpu.sync_copy(rows_v, acc.at[idx_v.at[j]], add=True)

    plsc.subcore_barrier()

    @pl.when(sid == 0)
    def _():
        pltpu.sync_copy(acc, p_hbm.at[cid])


def _sc_scatter(mij, dst2d, zeros):
    mesh = plsc.VectorSubcoreMesh(core_axis_name="c", subcore_axis_name="s",
                                  num_cores=NC, num_subcores=NS)
    kern = pl.kernel(
        _scatter_body,
        out_type=jax.ShapeDtypeStruct((NC, N_NODES, D), jnp.float32),
        mesh=mesh,
        compiler_params=_sc_compiler_params(),
        scratch_types=[
            pltpu.VMEM((SPAN, CHUNK), jnp.int32),
            pltpu.VMEM((CHUNK, D), jnp.float32),
            pltpu.VMEM_SHARED((ACC_ROWS, D), jnp.float32),
            pltpu.SemaphoreType.DMA,
        ],
    )
    return kern(mij, dst2d, zeros)


def _node_body(x_ref, p0_ref, p1_ref, w1x_ref, w1m_ref, b1_ref, w2_ref,
               b2_ref, out_ref):
    x = x_ref[...]
    mi = p0_ref[...] + p1_ref[...]
    t = _silu(jnp.dot(x, w1x_ref[...], preferred_element_type=jnp.float32,
                      precision=_HIGHEST)
              + jnp.dot(mi, w1m_ref[...], preferred_element_type=jnp.float32,
                        precision=_HIGHEST)
              + b1_ref[...])
    out_ref[...] = jnp.dot(t, w2_ref[...], preferred_element_type=jnp.float32,
                           precision=_HIGHEST) + b2_ref[...]


def _node_mlp(X, p0, p1, w1x, w1m, b1, w2, b2):
    bn = 1000
    return pl.pallas_call(
        _node_body,
        grid=(N_NODES // bn,),
        in_specs=[
            pl.BlockSpec((bn, D), lambda i: (i, 0)),
            pl.BlockSpec((bn, D), lambda i: (i, 0)),
            pl.BlockSpec((bn, D), lambda i: (i, 0)),
            pl.BlockSpec((D, D), lambda i: (0, 0)),
            pl.BlockSpec((D, D), lambda i: (0, 0)),
            pl.BlockSpec((1, D), lambda i: (0, 0)),
            pl.BlockSpec((D, D), lambda i: (0, 0)),
            pl.BlockSpec((1, D), lambda i: (0, 0)),
        ],
        out_specs=pl.BlockSpec((bn, D), lambda i: (i, 0)),
        out_shape=jax.ShapeDtypeStruct((N_NODES, D), jnp.float32),
    )(X, p0, p1, w1x, w1m, b1, w2, b2)


def kernel(X, E, emb_nodes, emb_edges, edge_index,
           W_e1, b_e1, W_e2, b_e2, W_h1, b_h1, W_h2, b_h2):
    src = edge_index[0]
    dst = edge_index[1]
    pad_rows = NCHUNK_PAD - NCHUNK
    src2d = jnp.pad(src.reshape(NCHUNK, CHUNK), ((0, pad_rows), (0, 0)))
    dst2d = jnp.pad(dst.reshape(NCHUNK, CHUNK), ((0, pad_rows), (0, 0)))
    ep4 = jnp.pad(emb_nodes, ((0, 0), (0, 4 - emb_nodes.shape[1]))).reshape(-1)
    ee3 = emb_edges.reshape(N_EDGES // BE, BE, 2)

    w1a = W_e1[0:D]
    w1b = W_e1[D:2 * D]
    wn = W_e1[2 * D:2 * D + 1]
    wd = W_e1[2 * D + 1:]
    b1 = b_e1.reshape(1, D)
    b2 = b_e2.reshape(1, D)

    ts, td = _make_tables(X, w1a, w1b)
    gs, gd, n2 = _sc_gather(ts, td, src2d, dst2d, ep4)
    mij = _edge_mlp(gs, gd, n2, ee3, wn, wd, b1, W_e2, b2)
    zeros = jnp.zeros((ACC_ROWS, D), jnp.float32)
    parts = _sc_scatter(mij, dst2d, zeros)
    X_out = _node_mlp(X, parts[0], parts[1], W_h1[0:D], W_h1[D:],
                      b_h1.reshape(1, D), W_h2, b_h2.reshape(1, D))
    return (X_out, mij, emb_nodes, emb_edges)
